# baseline (speedup 1.0000x reference)
import functools

import jax
import jax.numpy as jnp
from jax import lax
from jax.experimental import pallas as pl
from jax.experimental.pallas import tpu as pltpu
from jax.experimental.pallas import tpu_sc as plsc

N = 10000
E = 320000
D = 128
H = D // 2

CHUNK = 64
CHUNKS_PER_TILE = 320
IDX_ROWS = CHUNKS_PER_TILE // 2
E_PAD = 16 * CHUNKS_PER_TILE * CHUNK
N_PAD = 10240
R_ACC = 5024


def _proj_body(x_ref, wk_ref, bk_ref, wq_ref, bq_ref, wv_ref, bv_ref,
               ws_ref, bias_ref, kt_ref, qvh_ref, skip_ref):
    xb = x_ref[...]
    kt_ref[...] = jnp.dot(xb, wk_ref[...], preferred_element_type=jnp.float32) + bk_ref[...]
    qb = jnp.dot(xb, wq_ref[...], preferred_element_type=jnp.float32) + bq_ref[...]
    vb = jnp.dot(xb, wv_ref[...], preferred_element_type=jnp.float32) + bv_ref[...]
    qvh_ref[0, :, :H] = qb[:, :H]
    qvh_ref[0, :, H:] = vb[:, :H]
    qvh_ref[1, :, :H] = qb[:, H:]
    qvh_ref[1, :, H:] = vb[:, H:]
    skip_ref[...] = jnp.dot(xb, ws_ref[...], preferred_element_type=jnp.float32) + bias_ref[...]


def _combine_body(agg_ref, skip_ref, out_ref):
    a0 = agg_ref[0]
    a1 = agg_ref[1]
    out_ref[...] = jnp.concatenate(
        [a0[:, :H], a1[:, :H], a0[:, H:], a1[:, H:]], axis=1) + skip_ref[...]


def _sc_body(kt_hbm, qvh_hbm, dst_hbm, src_hbm, out_hbm,
             dst_v, src_v, kd_v, qv_v, msg_v, msg2_v, dsth_v, dsth2_v, par_v,
             agg_sh, sem_k, sem_qv, sem_s, sem_s2):
    c = lax.axis_index("c")
    s = lax.axis_index("s")

    def _zrow(i, _):
        for j in range(D // 16):
            msg_v[i, pl.ds(j * 16, 16)] = jnp.zeros((16,), jnp.float32)
        return 0
    lax.fori_loop(0, CHUNK, _zrow, 0, unroll=False)
    nzero = R_ACC // 32
    def _zcopy(t, _):
        ci = t * 16 + s
        @pl.when(ci < nzero)
        def _():
            pltpu.sync_copy(msg_v.at[pl.ds(0, 32)],
                            agg_sh.at[pl.ds(ci * 32, 32)])
        return 0
    lax.fori_loop(0, (nzero + 15) // 16, _zcopy, 0, unroll=False)

    pltpu.sync_copy(dst_hbm.at[s], dst_v)
    pltpu.sync_copy(src_hbm.at[s], src_v)

    plsc.subcore_barrier()

    def _dslice(ref, ci):
        return ref.at[ci >> 1, pl.ds((ci & 1) * CHUNK, CHUNK)]

    pltpu.async_copy(kt_hbm.at[_dslice(dst_v, 0)], kd_v.at[0], sem_k.at[0])
    pltpu.async_copy(qvh_hbm.at[c].at[_dslice(src_v, 0)], qv_v.at[0],
                     sem_qv.at[0])

    def _chunk(ci, _):
        buf = ci & 1
        nbuf = (ci + 1) & 1

        @pl.when(ci + 1 < CHUNKS_PER_TILE)
        def _():
            pltpu.async_copy(kt_hbm.at[_dslice(dst_v, ci + 1)],
                             kd_v.at[nbuf], sem_k.at[nbuf])
            pltpu.async_copy(qvh_hbm.at[c].at[_dslice(src_v, ci + 1)],
                             qv_v.at[nbuf], sem_qv.at[nbuf])

        pltpu.make_async_copy(kt_hbm.at[_dslice(dst_v, ci)], kd_v.at[buf],
                              sem_k.at[buf]).wait()
        pltpu.make_async_copy(qvh_hbm.at[c].at[_dslice(src_v, ci)],
                              qv_v.at[buf], sem_qv.at[buf]).wait()

        def _work(msg_r, dsth_r, sem_r):
            @pl.when(ci >= 2)
            def _():
                pltpu.make_async_copy(msg_r, agg_sh.at[dsth_r], sem_r).wait()

            def _pidx(i, _):
                d16 = dst_v[ci >> 1, pl.ds((ci & 1) * CHUNK + i * 16, 16)]
                dsth_r[pl.ds(i * 16, 16)] = lax.shift_right_logical(d16, 1)
                par_v[pl.ds(i * 16, 16)] = (d16 & 1).astype(jnp.float32)
                return 0
            lax.fori_loop(0, CHUNK // 16, _pidx, 0, unroll=False)

            @plsc.parallel_loop(0, CHUNK, unroll=8)
            def _row(e):
                bb = plsc.load_gather(par_v, [jnp.broadcast_to(e, (16,))])
                om = 1.0 - bb
                for j in range(H // 16):
                    kk = kd_v[buf, e, pl.ds(c * H + j * 16, 16)]
                    qq = qv_v[buf, e, pl.ds(j * 16, 16)]
                    vv = qv_v[buf, e, pl.ds(H + j * 16, 16)]
                    mh = vv / (1.0 + jnp.exp(-(kk + qq)))
                    msg_r[e, pl.ds(j * 16, 16)] = mh * om
                    msg_r[e, pl.ds(H + j * 16, 16)] = mh * bb

            pltpu.async_copy(msg_r, agg_sh.at[dsth_r], sem_r, add=True)

        @pl.when(buf == 0)
        def _():
            _work(msg_v, dsth_v, sem_s)

        @pl.when(buf == 1)
        def _():
            _work(msg2_v, dsth2_v, sem_s2)
        return 0
    lax.fori_loop(0, CHUNKS_PER_TILE, _chunk, 0, unroll=False)

    pltpu.make_async_copy(msg_v, agg_sh.at[dsth_v], sem_s).wait()
    pltpu.make_async_copy(msg2_v, agg_sh.at[dsth2_v], sem_s2).wait()

    plsc.subcore_barrier()

    ncopy = (N // 2) // 40
    def _ocopy(t, _):
        ci = t * 16 + s
        @pl.when(ci < ncopy)
        def _():
            pltpu.sync_copy(agg_sh.at[pl.ds(ci * 40, 40)],
                            out_hbm.at[c, pl.ds(ci * 40, 40)])
        return 0
    lax.fori_loop(0, (ncopy + 15) // 16, _ocopy, 0, unroll=False)


_sc_call = functools.partial(
    pl.kernel,
    mesh=plsc.VectorSubcoreMesh(core_axis_name="c", subcore_axis_name="s"),
    out_type=jax.ShapeDtypeStruct((2, N // 2, D), jnp.float32),
    compiler_params=pltpu.CompilerParams(needs_layout_passes=False),
    scratch_types=[
        pltpu.VMEM((IDX_ROWS, 128), jnp.int32),
        pltpu.VMEM((IDX_ROWS, 128), jnp.int32),
        pltpu.VMEM((2, CHUNK, D), jnp.float32),
        pltpu.VMEM((2, CHUNK, 2 * H), jnp.float32),
        pltpu.VMEM((CHUNK, D), jnp.float32),
        pltpu.VMEM((CHUNK, D), jnp.float32),
        pltpu.VMEM((CHUNK,), jnp.int32),
        pltpu.VMEM((CHUNK,), jnp.int32),
        pltpu.VMEM((CHUNK,), jnp.float32),
        pltpu.VMEM_SHARED((R_ACC, D), jnp.float32),
        pltpu.SemaphoreType.DMA((2,)),
        pltpu.SemaphoreType.DMA((2,)),
        pltpu.SemaphoreType.DMA,
        pltpu.SemaphoreType.DMA,
    ],
)(_sc_body)


@jax.jit
def kernel(x, edge_index, Wk, bk, Wq, bq, Wv, bv, Ws, bias):
    src = edge_index[0]
    dst = edge_index[1]
    npad = E_PAD - E
    src_p = jnp.concatenate([src, jnp.zeros((npad,), jnp.int32)])
    dst_p = jnp.concatenate(
        [dst, N + (jnp.arange(npad, dtype=jnp.int32) % (2 * (R_ACC - N // 2)))])
    src_t = src_p.reshape(16, IDX_ROWS, 128)
    dst_t = dst_p.reshape(16, IDX_ROWS, 128)

    xp = jnp.pad(x, ((0, N_PAD - N), (0, 0)))

    rb = 1024
    kt, qvh, skip = pl.pallas_call(
        _proj_body,
        grid=(N_PAD // rb,),
        in_specs=[
            pl.BlockSpec((rb, D), lambda i: (i, 0)),
            pl.BlockSpec((D, D), lambda i: (0, 0)),
            pl.BlockSpec((1, D), lambda i: (0, 0)),
            pl.BlockSpec((D, D), lambda i: (0, 0)),
            pl.BlockSpec((1, D), lambda i: (0, 0)),
            pl.BlockSpec((D, D), lambda i: (0, 0)),
            pl.BlockSpec((1, D), lambda i: (0, 0)),
            pl.BlockSpec((D, D), lambda i: (0, 0)),
            pl.BlockSpec((1, D), lambda i: (0, 0)),
        ],
        out_specs=[
            pl.BlockSpec((rb, D), lambda i: (i, 0)),
            pl.BlockSpec((2, rb, D), lambda i: (0, i, 0)),
            pl.BlockSpec((rb, D), lambda i: (i, 0)),
        ],
        out_shape=[
            jax.ShapeDtypeStruct((N_PAD, D), jnp.float32),
            jax.ShapeDtypeStruct((2, N_PAD, D), jnp.float32),
            jax.ShapeDtypeStruct((N_PAD, D), jnp.float32),
        ],
    )(xp, Wk, bk.reshape(1, D), Wq, bq.reshape(1, D),
      Wv, bv.reshape(1, D), Ws, bias.reshape(1, D))

    agg = _sc_call(kt, qvh, dst_t, src_t)

    rb2 = 1000
    out = pl.pallas_call(
        _combine_body,
        grid=((N // 2) // rb2,),
        in_specs=[
            pl.BlockSpec((2, rb2, D), lambda i: (0, i, 0)),
            pl.BlockSpec((rb2, 2 * D), lambda i: (i, 0)),
        ],
        out_specs=pl.BlockSpec((rb2, 2 * D), lambda i: (i, 0)),
        out_shape=jax.ShapeDtypeStruct((N // 2, 2 * D), jnp.float32),
    )(agg, skip.reshape(N_PAD // 2, 2 * D))
    return out.reshape(N, D)


# speedup vs baseline: 1.0012x; 1.0012x over previous
import functools

import jax
import jax.numpy as jnp
from jax import lax
from jax.experimental import pallas as pl
from jax.experimental.pallas import tpu as pltpu
from jax.experimental.pallas import tpu_sc as plsc

N = 10000
E = 320000
D = 128
H = D // 2

CHUNK = 64
CHUNKS_PER_TILE = 320
IDX_ROWS = CHUNKS_PER_TILE // 2
E_PAD = 16 * CHUNKS_PER_TILE * CHUNK
N_PAD = 10240
R_ACC = 5024


def _proj_body(x_ref, wk_ref, bk_ref, wq_ref, bq_ref, wv_ref, bv_ref,
               ws_ref, bias_ref, kt_ref, qvh_ref, skip_ref):
    xb = x_ref[...]
    kt_ref[...] = jnp.dot(xb, wk_ref[...], preferred_element_type=jnp.float32) + bk_ref[...]
    qb = jnp.dot(xb, wq_ref[...], preferred_element_type=jnp.float32) + bq_ref[...]
    vb = jnp.dot(xb, wv_ref[...], preferred_element_type=jnp.float32) + bv_ref[...]
    qvh_ref[0, :, :H] = qb[:, :H]
    qvh_ref[0, :, H:] = vb[:, :H]
    qvh_ref[1, :, :H] = qb[:, H:]
    qvh_ref[1, :, H:] = vb[:, H:]
    skip_ref[...] = jnp.dot(xb, ws_ref[...], preferred_element_type=jnp.float32) + bias_ref[...]


def _combine_body(agg_ref, skip_ref, out_ref):
    a0 = agg_ref[0]
    a1 = agg_ref[1]
    out_ref[...] = jnp.concatenate(
        [a0[:, :H], a1[:, :H], a0[:, H:], a1[:, H:]], axis=1) + skip_ref[...]


def _sc_body(kt_hbm, qvh_hbm, dst_hbm, src_hbm, out_hbm,
             dst_v, src_v, kd_v, qv_v, msg_v, msg2_v, dsth_v, dsth2_v, par_v,
             agg_sh, sem_k, sem_qv, sem_s, sem_s2):
    c = lax.axis_index("c")
    s = lax.axis_index("s")

    def _zrow(i, _):
        for j in range(D // 16):
            msg_v[i, pl.ds(j * 16, 16)] = jnp.zeros((16,), jnp.float32)
        return 0
    lax.fori_loop(0, CHUNK, _zrow, 0, unroll=False)
    nzero = R_ACC // 32
    def _zcopy(t, _):
        ci = t * 16 + s
        @pl.when(ci < nzero)
        def _():
            pltpu.sync_copy(msg_v.at[pl.ds(0, 32)],
                            agg_sh.at[pl.ds(ci * 32, 32)])
        return 0
    lax.fori_loop(0, (nzero + 15) // 16, _zcopy, 0, unroll=False)

    pltpu.sync_copy(dst_hbm.at[s], dst_v)
    pltpu.sync_copy(src_hbm.at[s], src_v)

    plsc.subcore_barrier()

    HC = CHUNK // 2

    def _dslice(ref, ci, h):
        return ref.at[ci >> 1, pl.ds((ci & 1) * CHUNK + h * HC, HC)]

    def _issue(ci, b):
        for h in range(2):
            pltpu.async_copy(kt_hbm.at[_dslice(dst_v, ci, h)],
                             kd_v.at[b, pl.ds(h * HC, HC)], sem_k.at[b, h])
            pltpu.async_copy(qvh_hbm.at[c].at[_dslice(src_v, ci, h)],
                             qv_v.at[b, pl.ds(h * HC, HC)], sem_qv.at[b, h])

    def _wait(ci, b):
        for h in range(2):
            pltpu.make_async_copy(kt_hbm.at[_dslice(dst_v, ci, h)],
                                  kd_v.at[b, pl.ds(h * HC, HC)],
                                  sem_k.at[b, h]).wait()
            pltpu.make_async_copy(qvh_hbm.at[c].at[_dslice(src_v, ci, h)],
                                  qv_v.at[b, pl.ds(h * HC, HC)],
                                  sem_qv.at[b, h]).wait()

    _issue(0, 0)

    def _chunk(ci, _):
        buf = ci & 1
        nbuf = (ci + 1) & 1

        @pl.when(ci + 1 < CHUNKS_PER_TILE)
        def _():
            _issue(ci + 1, nbuf)

        _wait(ci, buf)

        def _work(msg_r, dsth_r, sem_r):
            @pl.when(ci >= 2)
            def _():
                pltpu.make_async_copy(msg_r, agg_sh.at[dsth_r], sem_r).wait()

            for i in range(CHUNK // 16):
                d16 = dst_v[ci >> 1, pl.ds((ci & 1) * CHUNK + i * 16, 16)]
                dsth_r[pl.ds(i * 16, 16)] = lax.shift_right_logical(d16, 1)
                par_v[pl.ds(i * 16, 16)] = (d16 & 1).astype(jnp.float32)

            @plsc.parallel_loop(0, CHUNK, unroll=4)
            def _row(e):
                bb = plsc.load_gather(par_v, [jnp.broadcast_to(e, (16,))])
                om = 1.0 - bb
                for j in range(H // 16):
                    kk = kd_v[buf, e, pl.ds(c * H + j * 16, 16)]
                    qq = qv_v[buf, e, pl.ds(j * 16, 16)]
                    vv = qv_v[buf, e, pl.ds(H + j * 16, 16)]
                    mh = vv / (1.0 + jnp.exp(-(kk + qq)))
                    msg_r[e, pl.ds(j * 16, 16)] = mh * om
                    msg_r[e, pl.ds(H + j * 16, 16)] = mh * bb

            pltpu.async_copy(msg_r, agg_sh.at[dsth_r], sem_r, add=True)

        @pl.when(buf == 0)
        def _():
            _work(msg_v, dsth_v, sem_s)

        @pl.when(buf == 1)
        def _():
            _work(msg2_v, dsth2_v, sem_s2)
        return 0
    lax.fori_loop(0, CHUNKS_PER_TILE, _chunk, 0, unroll=False)

    pltpu.make_async_copy(msg_v, agg_sh.at[dsth_v], sem_s).wait()
    pltpu.make_async_copy(msg2_v, agg_sh.at[dsth2_v], sem_s2).wait()

    plsc.subcore_barrier()

    ncopy = (N // 2) // 40
    def _ocopy(t, _):
        ci = t * 16 + s
        @pl.when(ci < ncopy)
        def _():
            pltpu.sync_copy(agg_sh.at[pl.ds(ci * 40, 40)],
                            out_hbm.at[c, pl.ds(ci * 40, 40)])
        return 0
    lax.fori_loop(0, (ncopy + 15) // 16, _ocopy, 0, unroll=False)


_sc_call = functools.partial(
    pl.kernel,
    mesh=plsc.VectorSubcoreMesh(core_axis_name="c", subcore_axis_name="s"),
    out_type=jax.ShapeDtypeStruct((2, N // 2, D), jnp.float32),
    compiler_params=pltpu.CompilerParams(needs_layout_passes=False),
    scratch_types=[
        pltpu.VMEM((IDX_ROWS, 128), jnp.int32),
        pltpu.VMEM((IDX_ROWS, 128), jnp.int32),
        pltpu.VMEM((2, CHUNK, D), jnp.float32),
        pltpu.VMEM((2, CHUNK, 2 * H), jnp.float32),
        pltpu.VMEM((CHUNK, D), jnp.float32),
        pltpu.VMEM((CHUNK, D), jnp.float32),
        pltpu.VMEM((CHUNK,), jnp.int32),
        pltpu.VMEM((CHUNK,), jnp.int32),
        pltpu.VMEM((CHUNK,), jnp.float32),
        pltpu.VMEM_SHARED((R_ACC, D), jnp.float32),
        pltpu.SemaphoreType.DMA((2, 2)),
        pltpu.SemaphoreType.DMA((2, 2)),
        pltpu.SemaphoreType.DMA,
        pltpu.SemaphoreType.DMA,
    ],
)(_sc_body)


@jax.jit
def kernel(x, edge_index, Wk, bk, Wq, bq, Wv, bv, Ws, bias):
    src = edge_index[0]
    dst = edge_index[1]
    npad = E_PAD - E
    src_p = jnp.concatenate([src, jnp.zeros((npad,), jnp.int32)])
    dst_p = jnp.concatenate(
        [dst, N + (jnp.arange(npad, dtype=jnp.int32) % (2 * (R_ACC - N // 2)))])
    src_t = src_p.reshape(16, IDX_ROWS, 128)
    dst_t = dst_p.reshape(16, IDX_ROWS, 128)

    xp = jnp.pad(x, ((0, N_PAD - N), (0, 0)))

    rb = 1024
    kt, qvh, skip = pl.pallas_call(
        _proj_body,
        grid=(N_PAD // rb,),
        in_specs=[
            pl.BlockSpec((rb, D), lambda i: (i, 0)),
            pl.BlockSpec((D, D), lambda i: (0, 0)),
            pl.BlockSpec((1, D), lambda i: (0, 0)),
            pl.BlockSpec((D, D), lambda i: (0, 0)),
            pl.BlockSpec((1, D), lambda i: (0, 0)),
            pl.BlockSpec((D, D), lambda i: (0, 0)),
            pl.BlockSpec((1, D), lambda i: (0, 0)),
            pl.BlockSpec((D, D), lambda i: (0, 0)),
            pl.BlockSpec((1, D), lambda i: (0, 0)),
        ],
        out_specs=[
            pl.BlockSpec((rb, D), lambda i: (i, 0)),
            pl.BlockSpec((2, rb, D), lambda i: (0, i, 0)),
            pl.BlockSpec((rb, D), lambda i: (i, 0)),
        ],
        out_shape=[
            jax.ShapeDtypeStruct((N_PAD, D), jnp.float32),
            jax.ShapeDtypeStruct((2, N_PAD, D), jnp.float32),
            jax.ShapeDtypeStruct((N_PAD, D), jnp.float32),
        ],
    )(xp, Wk, bk.reshape(1, D), Wq, bq.reshape(1, D),
      Wv, bv.reshape(1, D), Ws, bias.reshape(1, D))

    agg = _sc_call(kt, qvh, dst_t, src_t)

    rb2 = 1000
    out = pl.pallas_call(
        _combine_body,
        grid=((N // 2) // rb2,),
        in_specs=[
            pl.BlockSpec((2, rb2, D), lambda i: (0, i, 0)),
            pl.BlockSpec((rb2, 2 * D), lambda i: (i, 0)),
        ],
        out_specs=pl.BlockSpec((rb2, 2 * D), lambda i: (i, 0)),
        out_shape=jax.ShapeDtypeStruct((N // 2, 2 * D), jnp.float32),
    )(agg, skip.reshape(N_PAD // 2, 2 * D))
    return out.reshape(N, D)


# speedup vs baseline: 1.0048x; 1.0036x over previous
import functools

import jax
import jax.numpy as jnp
from jax import lax
from jax.experimental import pallas as pl
from jax.experimental.pallas import tpu as pltpu
from jax.experimental.pallas import tpu_sc as plsc

N = 10000
E = 320000
D = 128
H = D // 2

CHUNK = 64
CHUNKS_PER_TILE = 320
IDX_ROWS = CHUNKS_PER_TILE // 2
E_PAD = 16 * CHUNKS_PER_TILE * CHUNK
N_PAD = 10240
R_ACC = 5024


def _proj_body(x_ref, wk_ref, bk_ref, wq_ref, bq_ref, wv_ref, bv_ref,
               ws_ref, bias_ref, kt_ref, qvh_ref, skip_ref):
    xb = x_ref[...]
    kt_ref[...] = jnp.dot(xb, wk_ref[...], preferred_element_type=jnp.float32) + bk_ref[...]
    qb = jnp.dot(xb, wq_ref[...], preferred_element_type=jnp.float32) + bq_ref[...]
    vb = jnp.dot(xb, wv_ref[...], preferred_element_type=jnp.float32) + bv_ref[...]
    qvh_ref[0, :, :H] = qb[:, :H]
    qvh_ref[0, :, H:] = vb[:, :H]
    qvh_ref[1, :, :H] = qb[:, H:]
    qvh_ref[1, :, H:] = vb[:, H:]
    skip_ref[...] = jnp.dot(xb, ws_ref[...], preferred_element_type=jnp.float32) + bias_ref[...]


def _combine_body(agg_ref, skip_ref, out_ref):
    a0 = agg_ref[0]
    a1 = agg_ref[1]
    out_ref[...] = jnp.concatenate(
        [a0[:, :H], a1[:, :H], a0[:, H:], a1[:, H:]], axis=1) + skip_ref[...]


def _sc_body(kt_hbm, qvh_hbm, dst_hbm, src_hbm, out_hbm,
             dst_v, src_v, kd_v, qv_v, msg_v, msg2_v, dsth_v, dsth2_v, par_v,
             agg_sh, sem_k, sem_qv, sem_s, sem_s2):
    c = lax.axis_index("c")
    s = lax.axis_index("s")

    def _zrow(i, _):
        for j in range(D // 16):
            msg_v[i, pl.ds(j * 16, 16)] = jnp.zeros((16,), jnp.float32)
        return 0
    lax.fori_loop(0, CHUNK, _zrow, 0, unroll=False)
    nzero = R_ACC // 32
    def _zcopy(t, _):
        ci = t * 16 + s
        @pl.when(ci < nzero)
        def _():
            pltpu.sync_copy(msg_v.at[pl.ds(0, 32)],
                            agg_sh.at[pl.ds(ci * 32, 32)])
        return 0
    lax.fori_loop(0, (nzero + 15) // 16, _zcopy, 0, unroll=False)

    pltpu.sync_copy(dst_hbm.at[s], dst_v)
    pltpu.sync_copy(src_hbm.at[s], src_v)

    plsc.subcore_barrier()

    def _dslice(ref, ci):
        return ref.at[ci >> 1, pl.ds((ci & 1) * CHUNK, CHUNK)]

    pltpu.async_copy(kt_hbm.at[_dslice(dst_v, 0)], kd_v.at[0], sem_k.at[0])
    pltpu.async_copy(qvh_hbm.at[c].at[_dslice(src_v, 0)], qv_v.at[0],
                     sem_qv.at[0])

    def _chunk(ci, _):
        buf = ci & 1
        nbuf = (ci + 1) & 1

        @pl.when(ci + 1 < CHUNKS_PER_TILE)
        def _():
            pltpu.async_copy(kt_hbm.at[_dslice(dst_v, ci + 1)],
                             kd_v.at[nbuf], sem_k.at[nbuf])
            pltpu.async_copy(qvh_hbm.at[c].at[_dslice(src_v, ci + 1)],
                             qv_v.at[nbuf], sem_qv.at[nbuf])

        pltpu.make_async_copy(kt_hbm.at[_dslice(dst_v, ci)], kd_v.at[buf],
                              sem_k.at[buf]).wait()
        pltpu.make_async_copy(qvh_hbm.at[c].at[_dslice(src_v, ci)],
                              qv_v.at[buf], sem_qv.at[buf]).wait()

        def _work(msg_r, dsth_r, sem_r):
            @pl.when(ci >= 2)
            def _():
                pltpu.make_async_copy(msg_r, agg_sh.at[dsth_r], sem_r).wait()

            def _pidx(i, _):
                d16 = dst_v[ci >> 1, pl.ds((ci & 1) * CHUNK + i * 16, 16)]
                dsth_r[pl.ds(i * 16, 16)] = lax.shift_right_logical(d16, 1)
                par_v[pl.ds(i * 16, 16)] = (d16 & 1).astype(jnp.float32)
                return 0
            lax.fori_loop(0, CHUNK // 16, _pidx, 0, unroll=False)

            @plsc.parallel_loop(0, CHUNK, unroll=4)
            def _row(e):
                bb = plsc.load_gather(par_v, [jnp.broadcast_to(e, (16,))])
                om = 1.0 - bb
                for j in range(H // 16):
                    kk = kd_v[buf, e, pl.ds(c * H + j * 16, 16)]
                    qq = qv_v[buf, e, pl.ds(j * 16, 16)]
                    vv = qv_v[buf, e, pl.ds(H + j * 16, 16)]
                    mh = vv / (1.0 + jnp.exp(-(kk + qq)))
                    msg_r[e, pl.ds(j * 16, 16)] = mh * om
                    msg_r[e, pl.ds(H + j * 16, 16)] = mh * bb

            pltpu.async_copy(msg_r, agg_sh.at[dsth_r], sem_r, add=True)

        @pl.when(buf == 0)
        def _():
            _work(msg_v, dsth_v, sem_s)

        @pl.when(buf == 1)
        def _():
            _work(msg2_v, dsth2_v, sem_s2)
        return 0
    lax.fori_loop(0, CHUNKS_PER_TILE, _chunk, 0, unroll=False)

    pltpu.make_async_copy(msg_v, agg_sh.at[dsth_v], sem_s).wait()
    pltpu.make_async_copy(msg2_v, agg_sh.at[dsth2_v], sem_s2).wait()

    plsc.subcore_barrier()

    ncopy = (N // 2) // 40
    def _ocopy(t, _):
        ci = t * 16 + s
        @pl.when(ci < ncopy)
        def _():
            pltpu.sync_copy(agg_sh.at[pl.ds(ci * 40, 40)],
                            out_hbm.at[c, pl.ds(ci * 40, 40)])
        return 0
    lax.fori_loop(0, (ncopy + 15) // 16, _ocopy, 0, unroll=False)


_sc_call = functools.partial(
    pl.kernel,
    mesh=plsc.VectorSubcoreMesh(core_axis_name="c", subcore_axis_name="s"),
    out_type=jax.ShapeDtypeStruct((2, N // 2, D), jnp.float32),
    compiler_params=pltpu.CompilerParams(needs_layout_passes=False),
    scratch_types=[
        pltpu.VMEM((IDX_ROWS, 128), jnp.int32),
        pltpu.VMEM((IDX_ROWS, 128), jnp.int32),
        pltpu.VMEM((2, CHUNK, D), jnp.float32),
        pltpu.VMEM((2, CHUNK, 2 * H), jnp.float32),
        pltpu.VMEM((CHUNK, D), jnp.float32),
        pltpu.VMEM((CHUNK, D), jnp.float32),
        pltpu.VMEM((CHUNK,), jnp.int32),
        pltpu.VMEM((CHUNK,), jnp.int32),
        pltpu.VMEM((CHUNK,), jnp.float32),
        pltpu.VMEM_SHARED((R_ACC, D), jnp.float32),
        pltpu.SemaphoreType.DMA((2,)),
        pltpu.SemaphoreType.DMA((2,)),
        pltpu.SemaphoreType.DMA,
        pltpu.SemaphoreType.DMA,
    ],
)(_sc_body)


@jax.jit
def kernel(x, edge_index, Wk, bk, Wq, bq, Wv, bv, Ws, bias):
    src = edge_index[0]
    dst = edge_index[1]
    npad = E_PAD - E
    src_p = jnp.concatenate([src, jnp.zeros((npad,), jnp.int32)])
    dst_p = jnp.concatenate(
        [dst, N + (jnp.arange(npad, dtype=jnp.int32) % (2 * (R_ACC - N // 2)))])
    src_t = src_p.reshape(16, IDX_ROWS, 128)
    dst_t = dst_p.reshape(16, IDX_ROWS, 128)

    xp = jnp.pad(x, ((0, N_PAD - N), (0, 0)))

    rb = 1024
    kt, qvh, skip = pl.pallas_call(
        _proj_body,
        grid=(N_PAD // rb,),
        in_specs=[
            pl.BlockSpec((rb, D), lambda i: (i, 0)),
            pl.BlockSpec((D, D), lambda i: (0, 0)),
            pl.BlockSpec((1, D), lambda i: (0, 0)),
            pl.BlockSpec((D, D), lambda i: (0, 0)),
            pl.BlockSpec((1, D), lambda i: (0, 0)),
            pl.BlockSpec((D, D), lambda i: (0, 0)),
            pl.BlockSpec((1, D), lambda i: (0, 0)),
            pl.BlockSpec((D, D), lambda i: (0, 0)),
            pl.BlockSpec((1, D), lambda i: (0, 0)),
        ],
        out_specs=[
            pl.BlockSpec((rb, D), lambda i: (i, 0)),
            pl.BlockSpec((2, rb, D), lambda i: (0, i, 0)),
            pl.BlockSpec((rb, D), lambda i: (i, 0)),
        ],
        out_shape=[
            jax.ShapeDtypeStruct((N_PAD, D), jnp.float32),
            jax.ShapeDtypeStruct((2, N_PAD, D), jnp.float32),
            jax.ShapeDtypeStruct((N_PAD, D), jnp.float32),
        ],
    )(xp, Wk, bk.reshape(1, D), Wq, bq.reshape(1, D),
      Wv, bv.reshape(1, D), Ws, bias.reshape(1, D))

    agg = _sc_call(kt, qvh, dst_t, src_t)

    rb2 = 1000
    out = pl.pallas_call(
        _combine_body,
        grid=((N // 2) // rb2,),
        in_specs=[
            pl.BlockSpec((2, rb2, D), lambda i: (0, i, 0)),
            pl.BlockSpec((rb2, 2 * D), lambda i: (i, 0)),
        ],
        out_specs=pl.BlockSpec((rb2, 2 * D), lambda i: (i, 0)),
        out_shape=jax.ShapeDtypeStruct((N // 2, 2 * D), jnp.float32),
    )(agg, skip.reshape(N_PAD // 2, 2 * D))
    return out.reshape(N, D)
